# baseline (speedup 1.0000x reference)
import functools

import jax
import jax.numpy as jnp
from jax import lax
from jax.experimental import pallas as pl
from jax.experimental.pallas import tpu as pltpu
from jax.experimental.pallas import tpu_sc as plsc

MAX_LEN = 8192
HIDDEN_DIM = 1024

_INFO = plsc.get_sparse_core_info()
_NC = _INFO.num_cores
_NS = _INFO.num_subcores
_NW = _NC * _NS
_B_PER_W = MAX_LEN // _NW


def _copy_kernel(table_hbm, out_hbm, sem):
    wid = lax.axis_index("s") * _NC + lax.axis_index("c")
    base = wid * _B_PER_W
    pltpu.async_copy(table_hbm.at[pl.ds(base, _B_PER_W)],
                     out_hbm.at[pl.ds(base, _B_PER_W)], sem).wait()


def kernel(seq_len, pos_embedding):
    del seq_len
    kern = functools.partial(
        pl.kernel,
        mesh=plsc.VectorSubcoreMesh(core_axis_name="c", subcore_axis_name="s"),
        out_type=jax.ShapeDtypeStruct((MAX_LEN, HIDDEN_DIM), jnp.float32),
        scratch_types=[pltpu.SemaphoreType.DMA],
    )(_copy_kernel)
    return kern(pos_embedding)


# speedup vs baseline: 24.7306x; 24.7306x over previous
import functools

import jax
import jax.numpy as jnp
from jax import lax
from jax.experimental import pallas as pl
from jax.experimental.pallas import tpu as pltpu
from jax.experimental.pallas import tpu_sc as plsc

MAX_LEN = 8192
HIDDEN_DIM = 1024

_INFO = plsc.get_sparse_core_info()
_NC = _INFO.num_cores
_NS = _INFO.num_subcores
_L = _INFO.num_lanes
_NW = _NC * _NS
_B_PER_W = MAX_LEN // _NW
_CHUNK = 32
_NCHUNK = _B_PER_W // _CHUNK
_NBUF = 3


def _copy_kernel(table_hbm, out_hbm, rows_v, *sems):
    gsems, ssems = sems[:_NBUF], sems[_NBUF:]
    wid = lax.axis_index("s") * _NC + lax.axis_index("c")
    base = wid * _B_PER_W

    def gather(c):
        return pltpu.async_copy(
            table_hbm.at[pl.ds(base + c * _CHUNK, _CHUNK)],
            rows_v.at[c % _NBUF], gsems[c % _NBUF])

    gh = [None] * _NCHUNK
    sh = [None] * _NCHUNK
    for c in range(_NBUF):
        gh[c] = gather(c)
    for c in range(_NCHUNK):
        gh[c].wait()
        sh[c] = pltpu.async_copy(
            rows_v.at[c % _NBUF],
            out_hbm.at[pl.ds(base + c * _CHUNK, _CHUNK)], ssems[c % _NBUF])
        if c + _NBUF < _NCHUNK:
            sh[c].wait()
            gh[c + _NBUF] = gather(c + _NBUF)
    for c in range(_NCHUNK - _NBUF, _NCHUNK):
        sh[c].wait()


def kernel(seq_len, pos_embedding):
    del seq_len
    kern = functools.partial(
        pl.kernel,
        mesh=plsc.VectorSubcoreMesh(core_axis_name="c", subcore_axis_name="s"),
        out_type=jax.ShapeDtypeStruct((MAX_LEN, HIDDEN_DIM), jnp.float32),
        scratch_types=[
            pltpu.VMEM((_NBUF, _CHUNK, HIDDEN_DIM), jnp.float32),
        ] + [pltpu.SemaphoreType.DMA] * (2 * _NBUF),
    )(_copy_kernel)
    return kern(pos_embedding)


# speedup vs baseline: 24.7818x; 1.0021x over previous
import functools

import jax
import jax.numpy as jnp
from jax import lax
from jax.experimental import pallas as pl
from jax.experimental.pallas import tpu as pltpu
from jax.experimental.pallas import tpu_sc as plsc

MAX_LEN = 8192
HIDDEN_DIM = 1024

_INFO = plsc.get_sparse_core_info()
_NC = _INFO.num_cores
_NS = _INFO.num_subcores
_L = _INFO.num_lanes
_NW = _NC * _NS
_B_PER_W = MAX_LEN // _NW
_CHUNK = 16
_NCHUNK = _B_PER_W // _CHUNK
_NBUF = 6


def _copy_kernel(table_hbm, out_hbm, rows_v, *sems):
    gsems, ssems = sems[:_NBUF], sems[_NBUF:]
    wid = lax.axis_index("s") * _NC + lax.axis_index("c")
    base = wid * _B_PER_W

    def gather(c):
        return pltpu.async_copy(
            table_hbm.at[pl.ds(base + c * _CHUNK, _CHUNK)],
            rows_v.at[c % _NBUF], gsems[c % _NBUF])

    gh = [None] * _NCHUNK
    sh = [None] * _NCHUNK
    for c in range(_NBUF):
        gh[c] = gather(c)
    for c in range(_NCHUNK):
        gh[c].wait()
        sh[c] = pltpu.async_copy(
            rows_v.at[c % _NBUF],
            out_hbm.at[pl.ds(base + c * _CHUNK, _CHUNK)], ssems[c % _NBUF])
        if c + _NBUF < _NCHUNK:
            sh[c].wait()
            gh[c + _NBUF] = gather(c + _NBUF)
    for c in range(_NCHUNK - _NBUF, _NCHUNK):
        sh[c].wait()


def kernel(seq_len, pos_embedding):
    del seq_len
    kern = functools.partial(
        pl.kernel,
        mesh=plsc.VectorSubcoreMesh(core_axis_name="c", subcore_axis_name="s"),
        out_type=jax.ShapeDtypeStruct((MAX_LEN, HIDDEN_DIM), jnp.float32),
        scratch_types=[
            pltpu.VMEM((_NBUF, _CHUNK, HIDDEN_DIM), jnp.float32),
        ] + [pltpu.SemaphoreType.DMA] * (2 * _NBUF),
    )(_copy_kernel)
    return kern(pos_embedding)


# speedup vs baseline: 30.9666x; 1.2496x over previous
import jax
import jax.numpy as jnp
from jax.experimental import pallas as pl
from jax.experimental.pallas import tpu as pltpu

MAX_LEN = 8192
HIDDEN_DIM = 1024
_BR = 256


def _copy_body(in_ref, out_ref):
    out_ref[...] = in_ref[...]


def kernel(seq_len, pos_embedding):
    del seq_len
    return pl.pallas_call(
        _copy_body,
        grid=(MAX_LEN // _BR,),
        in_specs=[pl.BlockSpec((_BR, HIDDEN_DIM), lambda i: (i, 0))],
        out_specs=pl.BlockSpec((_BR, HIDDEN_DIM), lambda i: (i, 0)),
        out_shape=jax.ShapeDtypeStruct((MAX_LEN, HIDDEN_DIM), jnp.float32),
    )(pos_embedding)
